# baseline (speedup 1.0000x reference)
import functools

import jax
import jax.numpy as jnp
from jax import lax
from jax.experimental import pallas as pl
from jax.experimental.pallas import tpu as pltpu
from jax.experimental.pallas import tpu_sc as plsc

_N = 100000
_V = 120
_D = 256
_NW = 32
_CH = 112
_NB = 2
_NCH = 28
_BPW = _CH * _NCH
_G = _CH // 16


def _embed_body(idx_hbm, table_hbm, out_hbm, idx_v, table_v, buf0, buf1,
                wsem0, wsem1):
    bufs = (buf0, buf1)
    wsems = (wsem0, wsem1)
    wid = lax.axis_index("s") * 2 + lax.axis_index("c")
    base = jnp.minimum(wid * _BPW, _N - _BPW)

    pltpu.sync_copy(table_hbm, table_v)
    pltpu.sync_copy(idx_hbm.at[pl.ds(base, _BPW)], idx_v)

    def sub1(i, carry):
        idx_v[pl.ds(i * 16, 16)] = idx_v[pl.ds(i * 16, 16)] - 1
        return carry

    lax.fori_loop(0, _BPW // 16, sub1, 0)

    lanes256 = lax.iota(jnp.int32, 16) * _D
    sas0 = [g * 16 * _D + lanes256 for g in range(_G)]

    def compute(c, b):
        las0 = [idx_v[pl.ds(c * _CH + g * 16, 16)] * _D for g in range(_G)]

        def pbody(p, carry):
            las, sas = carry
            for g in range(_G):
                val = plsc.load_gather(table_v, [las[g]])
                plsc.store_scatter(bufs[b], [sas[g]], val)
            return ([a + 1 for a in las], [a + 1 for a in sas])

        lax.fori_loop(0, _D, pbody, (las0, sas0))

    def start_write(c, b):
        pltpu.make_async_copy(
            bufs[b], out_hbm.at[pl.ds((base + c * _CH) * _D, _CH * _D)],
            wsems[b]
        ).start()

    def wait_write(b):
        pltpu.make_async_copy(
            bufs[b], out_hbm.at[pl.ds(base * _D, _CH * _D)], wsems[b]
        ).wait()

    for b in range(_NB):
        compute(b, b)
        start_write(b, b)

    def body(c2, carry):
        for b in range(_NB):
            c = (c2 + 1) * _NB + b
            wait_write(b)
            compute(c, b)
            start_write(c, b)
        return carry

    lax.fori_loop(0, _NCH // _NB - 1, body, 0)

    for b in range(_NB):
        wait_write(b)


@jax.jit
def _embed_lookup(idx, table_flat):
    mesh = plsc.VectorSubcoreMesh(core_axis_name="c", subcore_axis_name="s")
    fn = pl.kernel(
        _embed_body,
        mesh=mesh,
        compiler_params=pltpu.CompilerParams(needs_layout_passes=False),
        out_type=jax.ShapeDtypeStruct((_N * _D,), jnp.float32),
        scratch_types=(
            [pltpu.VMEM((_BPW,), jnp.int32),
             pltpu.VMEM((_V * _D,), jnp.float32)]
            + [pltpu.VMEM((_CH * _D,), jnp.float32) for _ in range(_NB)]
            + [pltpu.SemaphoreType.DMA for _ in range(_NB)]
        ),
    )
    return fn(idx, table_flat)


def kernel(atomic_numbers, atom_embedding_weight):
    out = _embed_lookup(atomic_numbers, atom_embedding_weight.reshape(-1))
    return out.reshape(_N, _D)


# speedup vs baseline: 2.1901x; 2.1901x over previous
import functools

import jax
import jax.numpy as jnp
from jax import lax
from jax.experimental import pallas as pl
from jax.experimental.pallas import tpu as pltpu
from jax.experimental.pallas import tpu_sc as plsc

_N = 100000
_V = 120
_D = 256
_NW = 32
_CH = 112
_NB = 2
_NCH = 28
_BPW = _CH * _NCH
_G = _CH // 16


def _embed_body(idx_hbm, table_hbm, out_hbm, idx_v, table_v, buf0, buf1,
                wsem0, wsem1):
    bufs = (buf0, buf1)
    wsems = (wsem0, wsem1)
    wid = lax.axis_index("s") * 2 + lax.axis_index("c")
    base = jnp.minimum(wid * _BPW, _N - _BPW)

    pltpu.sync_copy(table_hbm, table_v)
    pltpu.sync_copy(idx_hbm.at[pl.ds(base, _BPW)], idx_v)

    def sub1(i, carry):
        idx_v[pl.ds(i * 16, 16)] = idx_v[pl.ds(i * 16, 16)] - 1
        return carry

    lax.fori_loop(0, _BPW // 16, sub1, 0)

    lanes256 = lax.iota(jnp.int32, 16) * _D
    sas0 = [g * 16 * _D + lanes256 for g in range(_G)]

    def compute(c, b):
        las0 = [idx_v[pl.ds(c * _CH + g * 16, 16)] * _D for g in range(_G)]

        @plsc.parallel_loop(0, _D, unroll=8, carry=(las0, sas0))
        def _pbody(p, carry):
            las, sas = carry
            for g in range(_G):
                val = plsc.load_gather(table_v, [las[g]])
                plsc.store_scatter(bufs[b], [sas[g]], val)
            return ([a + 1 for a in las], [a + 1 for a in sas])

    def start_write(c, b):
        pltpu.make_async_copy(
            bufs[b], out_hbm.at[pl.ds((base + c * _CH) * _D, _CH * _D)],
            wsems[b]
        ).start()

    def wait_write(b):
        pltpu.make_async_copy(
            bufs[b], out_hbm.at[pl.ds(base * _D, _CH * _D)], wsems[b]
        ).wait()

    for b in range(_NB):
        compute(b, b)
        start_write(b, b)

    def body(c2, carry):
        for b in range(_NB):
            c = (c2 + 1) * _NB + b
            wait_write(b)
            compute(c, b)
            start_write(c, b)
        return carry

    lax.fori_loop(0, _NCH // _NB - 1, body, 0)

    for b in range(_NB):
        wait_write(b)


@jax.jit
def _embed_lookup(idx, table_flat):
    mesh = plsc.VectorSubcoreMesh(core_axis_name="c", subcore_axis_name="s")
    fn = pl.kernel(
        _embed_body,
        mesh=mesh,
        compiler_params=pltpu.CompilerParams(needs_layout_passes=False),
        out_type=jax.ShapeDtypeStruct((_N * _D,), jnp.float32),
        scratch_types=(
            [pltpu.VMEM((_BPW,), jnp.int32),
             pltpu.VMEM((_V * _D,), jnp.float32)]
            + [pltpu.VMEM((_CH * _D,), jnp.float32) for _ in range(_NB)]
            + [pltpu.SemaphoreType.DMA for _ in range(_NB)]
        ),
    )
    return fn(idx, table_flat)


def kernel(atomic_numbers, atom_embedding_weight):
    out = _embed_lookup(atomic_numbers, atom_embedding_weight.reshape(-1))
    return out.reshape(_N, _D)


# speedup vs baseline: 4.0816x; 1.8637x over previous
import jax
import jax.numpy as jnp
from jax import lax
from jax.experimental import pallas as pl
from jax.experimental.pallas import tpu as pltpu
from jax.experimental.pallas import tpu_sc as plsc

_N = 100000
_V = 120
_D = 256
_NW = 32
_CH = 112
_NB = 4
_NCH = 28
_BPW = _CH * _NCH


def _embed_body(idx_hbm, table_hbm, out_hbm, idx_v, buf0, buf1, buf2, buf3,
                gsem0, gsem1, gsem2, gsem3, wsem0, wsem1, wsem2, wsem3):
    bufs = (buf0, buf1, buf2, buf3)
    gsems = (gsem0, gsem1, gsem2, gsem3)
    wsems = (wsem0, wsem1, wsem2, wsem3)
    wid = lax.axis_index("s") * 2 + lax.axis_index("c")
    base = jnp.minimum(wid * _BPW, _N - _BPW)

    pltpu.sync_copy(idx_hbm.at[pl.ds(base, _BPW)], idx_v)

    def sub1(i, carry):
        idx_v[pl.ds(i * 16, 16)] = idx_v[pl.ds(i * 16, 16)] - 1
        return carry

    lax.fori_loop(0, _BPW // 16, sub1, 0)

    def start_gather(c, b):
        pltpu.make_async_copy(
            table_hbm.at[idx_v.at[pl.ds(c * _CH, _CH)]], bufs[b], gsems[b]
        ).start()

    def wait_gather(b):
        pltpu.make_async_copy(
            table_hbm.at[idx_v.at[pl.ds(0, _CH)]], bufs[b], gsems[b]
        ).wait()

    def start_write(c, b):
        pltpu.make_async_copy(
            bufs[b], out_hbm.at[pl.ds(base + c * _CH, _CH)], wsems[b]
        ).start()

    def wait_write(b):
        pltpu.make_async_copy(
            bufs[b], out_hbm.at[pl.ds(base, _CH)], wsems[b]
        ).wait()

    for b in range(_NB):
        start_gather(b, b)

    def body(c4, carry):
        for b in range(_NB):
            wait_gather(b)
            start_write(c4 * _NB + b, b)
        for b in range(_NB):
            wait_write(b)
            start_gather(c4 * _NB + _NB + b, b)
        return carry

    lax.fori_loop(0, _NCH // _NB - 1, body, 0)

    for b in range(_NB):
        wait_gather(b)
        start_write(_NCH - _NB + b, b)
    for b in range(_NB):
        wait_write(b)


@jax.jit
def _embed_lookup(idx, table):
    mesh = plsc.VectorSubcoreMesh(core_axis_name="c", subcore_axis_name="s")
    fn = pl.kernel(
        _embed_body,
        mesh=mesh,
        out_type=jax.ShapeDtypeStruct((_N, _D), jnp.float32),
        scratch_types=(
            [pltpu.VMEM((_BPW,), jnp.int32)]
            + [pltpu.VMEM((_CH, _D), jnp.float32) for _ in range(_NB)]
            + [pltpu.SemaphoreType.DMA for _ in range(2 * _NB)]
        ),
    )
    return fn(idx, table)


def kernel(atomic_numbers, atom_embedding_weight):
    return _embed_lookup(atomic_numbers, atom_embedding_weight)
